# baseline (speedup 1.0000x reference)
import functools

import jax
import jax.numpy as jnp
from jax import lax
from jax.experimental import pallas as pl
from jax.experimental.pallas import tpu as pltpu
from jax.experimental.pallas import tpu_sc as plsc

NC = 2
NS = 16
K = 100
KD = 80
BN = 1000
V_TH = 1.0
T = 4


def _deg_body(dst_hbm, zeros_hbm, degp_hbm, idx_v, ones_v, deg_sh, sem):
    nchunk = dst_hbm.shape[2]
    c = lax.axis_index("c")
    s = lax.axis_index("s")

    @pl.when(s == 0)
    def _():
        pltpu.sync_copy(zeros_hbm, deg_sh)

    for i in range(KD // 16):
        ones_v[pl.ds(i * 16, 16)] = jnp.ones((16,), jnp.float32)
    pltpu.sync_copy(dst_hbm.at[c, s], idx_v)
    plsc.subcore_barrier()

    def body(j, carry):
        pltpu.sync_copy(ones_v, deg_sh.at[idx_v.at[j]], add=True)
        return carry

    lax.fori_loop(0, nchunk, body, 0)
    plsc.subcore_barrier()

    @pl.when(s == 0)
    def _():
        pltpu.sync_copy(deg_sh, degp_hbm.at[c])


def _deg_partials(dst_r, zeros_n, n):
    nchunk = dst_r.shape[2]
    kern = pl.kernel(
        _deg_body,
        out_type=jax.ShapeDtypeStruct((NC, n), jnp.float32),
        mesh=plsc.VectorSubcoreMesh(core_axis_name="c", subcore_axis_name="s"),
        scratch_types=[
            pltpu.VMEM((nchunk, KD), jnp.int32),
            pltpu.VMEM((KD,), jnp.float32),
            pltpu.MemorySpace.VMEM_SHARED((n,), jnp.float32),
            pltpu.SemaphoreType.DMA,
        ],
    )
    return kern(dst_r, zeros_n)


def _scatter_body(g_hbm, src_hbm, dst_hbm, zeros_hbm, accp_hbm,
                  sidx_v, didx_a, didx_b, rows_a, rows_b, acc_sh,
                  sem_a, sem_b, sem_da, sem_db):
    nchunk = dst_hbm.shape[2] - 2
    n = zeros_hbm.shape[0]
    rpt = n // NS
    c = lax.axis_index("c")
    s = lax.axis_index("s")

    rpt8 = (rpt // 8) * 8
    tail = n - NS * rpt8

    def _spmem_slices(src_at, dst_at):
        pltpu.sync_copy(src_at(s * rpt8, rpt8), dst_at(s * rpt8, rpt8))
        if tail:
            @pl.when(s == 0)
            def _():
                pltpu.sync_copy(src_at(NS * rpt8, tail), dst_at(NS * rpt8, tail))

    _spmem_slices(lambda o, l: zeros_hbm.at[pl.ds(o, l)],
                  lambda o, l: acc_sh.at[pl.ds(o, l)])
    pltpu.sync_copy(src_hbm.at[c, s], sidx_v)
    plsc.subcore_barrier()

    pltpu.async_copy(g_hbm.at[sidx_v.at[0]], rows_a, sem_a)
    pltpu.async_copy(g_hbm.at[sidx_v.at[1]], rows_b, sem_b)
    pltpu.async_copy(dst_hbm.at[c, s, 0], didx_a, sem_da)
    pltpu.async_copy(dst_hbm.at[c, s, 1], didx_b, sem_db)

    def _step(a, rows, didx, sem, sem_d):
        pltpu.make_async_copy(g_hbm.at[sidx_v.at[a]], rows, sem).wait()
        pltpu.make_async_copy(dst_hbm.at[c, s, a], didx, sem_d).wait()
        pltpu.sync_copy(rows, acc_sh.at[didx], add=True)
        pltpu.async_copy(g_hbm.at[sidx_v.at[a + 2]], rows, sem)
        pltpu.async_copy(dst_hbm.at[c, s, a + 2], didx, sem_d)

    def body(i, carry):
        _step(2 * i, rows_a, didx_a, sem_a, sem_da)
        _step(2 * i + 1, rows_b, didx_b, sem_b, sem_db)
        return carry

    lax.fori_loop(0, nchunk // 2, body, 0)
    pltpu.make_async_copy(g_hbm.at[sidx_v.at[nchunk]], rows_a, sem_a).wait()
    pltpu.make_async_copy(g_hbm.at[sidx_v.at[nchunk + 1]], rows_b, sem_b).wait()
    pltpu.make_async_copy(dst_hbm.at[c, s, nchunk], didx_a, sem_da).wait()
    pltpu.make_async_copy(dst_hbm.at[c, s, nchunk + 1], didx_b, sem_db).wait()
    plsc.subcore_barrier()
    _spmem_slices(lambda o, l: acc_sh.at[pl.ds(o, l)],
                  lambda o, l: accp_hbm.at[c, pl.ds(o, l)])


def _scatter_partials(g, src_rp, dst_rp, zeros_nd, n, d):
    nchunk = dst_rp.shape[2] - 2
    kern = pl.kernel(
        _scatter_body,
        out_type=jax.ShapeDtypeStruct((NC, n, d), jnp.float32),
        mesh=plsc.VectorSubcoreMesh(core_axis_name="c", subcore_axis_name="s"),
        scratch_types=[
            pltpu.VMEM((nchunk + 2, K), jnp.int32),
            pltpu.VMEM((K,), jnp.int32),
            pltpu.VMEM((K,), jnp.int32),
            pltpu.VMEM((K, d), jnp.float32),
            pltpu.VMEM((K, d), jnp.float32),
            pltpu.MemorySpace.VMEM_SHARED((n, d), jnp.float32),
            pltpu.SemaphoreType.DMA,
            pltpu.SemaphoreType.DMA,
            pltpu.SemaphoreType.DMA,
            pltpu.SemaphoreType.DMA,
        ],
    )
    return kern(g, src_rp, dst_rp, zeros_nd)


def _dinv_of(degp_blk):
    deg = degp_blk[:, 0:1] + degp_blk[:, 1:2] + 1.0
    return lax.rsqrt(jnp.maximum(deg, 1e-12))


def _mm_scale_body(x_ref, w_ref, degp_ref, g_ref):
    h = jnp.dot(x_ref[...], w_ref[...], preferred_element_type=jnp.float32)
    g_ref[...] = _dinv_of(degp_ref[...]) * h


def _mm_scale(x, w, degp_t):
    n, din = x.shape
    dout = w.shape[1]
    return pl.pallas_call(
        _mm_scale_body,
        grid=(n // BN,),
        in_specs=[
            pl.BlockSpec((BN, din), lambda i: (i, 0)),
            pl.BlockSpec((din, dout), lambda i: (0, 0)),
            pl.BlockSpec((BN, NC), lambda i: (i, 0)),
        ],
        out_specs=pl.BlockSpec((BN, dout), lambda i: (i, 0)),
        out_shape=jax.ShapeDtypeStruct((n, dout), jnp.float32),
    )(x, w, degp_t)


def _if_body(accp_ref, g_ref, degp_ref, b_ref, o_ref, z_ref):
    dinv = _dinv_of(degp_ref[...])
    g = g_ref[...]
    y = dinv * (accp_ref[0] + accp_ref[1] + g) + b_ref[...]
    z = jnp.zeros_like(y)
    for t in range(T):
        z = z + y
        o = (z >= V_TH).astype(jnp.float32)
        z = z * (1.0 - o)
        o_ref[t] = o
        z_ref[t] = z


def _if_dynamics(accp, g, degp_t, b2d):
    n, d = g.shape
    out_sds = jax.ShapeDtypeStruct((T, n, d), jnp.float32)
    return pl.pallas_call(
        _if_body,
        grid=(n // BN,),
        in_specs=[
            pl.BlockSpec((NC, BN, d), lambda i: (0, i, 0)),
            pl.BlockSpec((BN, d), lambda i: (i, 0)),
            pl.BlockSpec((BN, NC), lambda i: (i, 0)),
            pl.BlockSpec((1, d), lambda i: (0, 0)),
        ],
        out_specs=[
            pl.BlockSpec((T, BN, d), lambda i: (0, i, 0)),
            pl.BlockSpec((T, BN, d), lambda i: (0, i, 0)),
        ],
        out_shape=[out_sds, out_sds],
    )(accp, g, degp_t, b2d)


def kernel(x, edge_index, W, b):
    n, din = x.shape
    dout = W.shape[1]
    e = edge_index.shape[1]
    ept = e // (NC * NS)
    nchunk = ept // K
    nchunk_d = ept // KD

    pad = jnp.zeros((NC, NS, 2, K), jnp.int32)
    src_rp = jnp.concatenate(
        [edge_index[0].reshape(NC, NS, nchunk, K), pad], axis=2)
    dst_rp = jnp.concatenate(
        [edge_index[1].reshape(NC, NS, nchunk, K), pad], axis=2)
    dst_rd = edge_index[1].reshape(NC, NS, nchunk_d, KD)
    zeros_n = jnp.zeros((n,), jnp.float32)
    zeros_nd = jnp.zeros((n, dout), jnp.float32)

    degp = _deg_partials(dst_rd, zeros_n, n)
    degp_t = degp.T
    g = _mm_scale(x, W, degp_t)
    accp = _scatter_partials(g, src_rp, dst_rp, zeros_nd, n, dout)
    o_seq, z_seq = _if_dynamics(accp, g, degp_t, b.reshape(1, dout))
    return (o_seq, z_seq)
